# baseline (speedup 1.0000x reference)
import jax
import jax.numpy as jnp
import numpy as np
from jax.experimental import pallas as pl
from jax.experimental.pallas import tpu as pltpu

B, S, D = 1, 2048, 768
H, DH = 12, 64
EA, KA = 8, 2
EF, KF, DF = 16, 8, 128

BLK = 512
NBLK = S // BLK

f32 = jnp.float32
bf16 = jnp.bfloat16


def _topk_gates_mask(logits, k):
    n, e_dim = logits.shape
    lj = logits[:, :, None]
    le = logits[:, None, :]
    jj = jax.lax.broadcasted_iota(jnp.int32, (n, e_dim, e_dim), 1)
    ee = jax.lax.broadcasted_iota(jnp.int32, (n, e_dim, e_dim), 2)
    gt = (lj > le).astype(f32)
    tie = jnp.logical_and(lj == le, jj < ee).astype(f32)
    rank = jnp.sum(gt + tie, axis=1)
    keep = (rank < k).astype(f32)
    return jax.nn.sigmoid(logits) * keep


def _rmsnorm(x, g, eps=1e-6):
    return x * jax.lax.rsqrt(jnp.mean(x * x, axis=-1, keepdims=True) + eps) * g


def _body(x_ref, ga_ref, gf_ref, cos_ref, sin_ref, teye_ref,
          wqk_ref, wv_ref, selv_ref, wo_ref, selo_ref, w1_ref, w2_ref, self_ref,
          out_ref, kt_scr, v_scr):
    i = pl.program_id(0)
    x = x_ref[...]
    h = _rmsnorm(x, ga_ref[...])
    hb = h.astype(bf16)

    hd = H * DH
    cs = jnp.dot(cos_ref[...], teye_ref[...], preferred_element_type=f32)
    sn = jnp.dot(sin_ref[...], teye_ref[...], preferred_element_type=f32)
    sq = jnp.dot(hb, wqk_ref[:, :2 * hd], preferred_element_type=f32)
    q = (sq[:, :hd] * cs + sq[:, hd:] * sn).astype(bf16)
    sk = jnp.dot(hb, wqk_ref[:, 2 * hd:], preferred_element_type=f32)
    k = (sk[:, :hd] * cs + sk[:, hd:] * sn).astype(bf16)

    ktf = k.T
    for hh in range(H):
        kt_scr[hh, i] = ktf[hh * DH:(hh + 1) * DH, :]

    logits = jnp.dot(h, selv_ref[...], preferred_element_type=f32)
    gates = _topk_gates_mask(logits, KA)
    acc = jnp.zeros((BLK, H * DH), f32)
    for e in range(EA):
        acc += jnp.dot(hb, wv_ref[e], preferred_element_type=f32) * gates[:, e:e + 1]
    v = acc.astype(bf16)
    for hh in range(H):
        v_scr[hh, i] = v[:, hh * DH:(hh + 1) * DH]

    rowg = i * BLK + jax.lax.broadcasted_iota(jnp.int32, (BLK, BLK), 0)
    coll = jax.lax.broadcasted_iota(jnp.int32, (BLK, BLK), 1)
    ctx_parts = []
    for hh in range(H):
        qh = q[:, hh * DH:(hh + 1) * DH]

        def chunk(c, carry):
            m, l, a = carry
            sc = jnp.dot(qh, kt_scr[hh, c], preferred_element_type=f32)
            colg = c * BLK + coll
            sc = jnp.where(colg <= rowg, sc, -1e9)
            m_new = jnp.maximum(m, jnp.max(sc, axis=-1, keepdims=True))
            corr = jnp.exp(m - m_new)
            p = jnp.exp(sc - m_new)
            l_new = l * corr + jnp.sum(p, axis=-1, keepdims=True)
            a_new = a * corr + jnp.dot(p.astype(bf16), v_scr[hh, c],
                                       preferred_element_type=f32)
            return m_new, l_new, a_new

        m0 = jnp.full((BLK, 1), -1e30, f32)
        l0 = jnp.zeros((BLK, 1), f32)
        a0 = jnp.zeros((BLK, DH), f32)
        m, l, a = jax.lax.fori_loop(0, i + 1, chunk, (m0, l0, a0))
        ctx_parts.append((a / l).astype(bf16))
    ctx = jnp.concatenate(ctx_parts, axis=1)

    logits_o = jnp.dot(h, selo_ref[...], preferred_element_type=f32)
    gates_o = _topk_gates_mask(logits_o, KA)
    acc = jnp.zeros((BLK, D), f32)
    for e in range(EA):
        acc += jnp.dot(ctx, wo_ref[e], preferred_element_type=f32) * gates_o[:, e:e + 1]
    x1 = x + acc

    h2 = _rmsnorm(x1, gf_ref[...])
    h2b = h2.astype(bf16)
    logits_f = jnp.dot(h2, self_ref[...], preferred_element_type=f32)
    gates_f = _topk_gates_mask(logits_f, KF)
    y = jnp.zeros((BLK, D), f32)
    lane = jax.lax.broadcasted_iota(jnp.int32, (BLK, 2 * DF), 1)
    for pp in range(EF // 2):
        mid = jnp.dot(h2b, w1_ref[pp], preferred_element_type=f32)
        mid = jnp.maximum(mid, 0.0)
        gw = jnp.where(lane < DF, gates_f[:, 2 * pp:2 * pp + 1],
                       gates_f[:, 2 * pp + 1:2 * pp + 2])
        mid = (mid * gw).astype(bf16)
        y += jnp.dot(mid, w2_ref[pp], preferred_element_type=f32)
    out_ref[...] = x1 + y


def _full(shape):
    return pl.BlockSpec(shape, lambda *_: (0,) * len(shape))


def _rope_tables():
    half = DH // 2
    pos = np.arange(S, dtype=np.float32)
    inv_freq = 1.0 / (10000.0 ** (np.arange(0, half, dtype=np.float32) / half))
    freqs = (pos[:, None] * inv_freq[None, :]).astype(np.float32)
    cos1 = np.cos(freqs, dtype=np.float32)
    sin1 = np.sin(freqs, dtype=np.float32)
    cos = np.tile(np.concatenate([cos1, cos1], axis=1), (1, H))
    sin = np.tile(np.concatenate([sin1, sin1], axis=1), (1, H))
    scale = np.float32(1.0) / np.sqrt(np.float32(DH))
    return (cos.astype(np.float32), sin.astype(np.float32),
            (cos * scale).astype(np.float32), (sin * scale).astype(np.float32))


_COS, _SIN, _COSQ, _SINQ = _rope_tables()
_COSC = _COS[:, :DH].copy()
_SINC = _SIN[:, :DH].copy()
_TEYE = np.tile(np.eye(DH, dtype=np.float32), (1, H))


def kernel(token_stream, g_attn, g_ffn, Wq, Wk, Wv, Wo, sel_v, sel_o, W1, W2, sel_f):
    x = token_stream[0]
    ga = g_attn.reshape(1, D)
    gf = g_ffn.reshape(1, D)

    def rot_w(w):
        wr = w.reshape(D, H, 2, DH // 2)
        return jnp.concatenate([-wr[:, :, 1], wr[:, :, 0]], axis=2).reshape(D, H * DH)

    wqk = jnp.concatenate([Wq * 0.125, rot_w(Wq) * 0.125, Wk, rot_w(Wk)],
                          axis=1).astype(bf16)
    wv, wo = Wv.astype(bf16), Wo.astype(bf16)
    w1 = (W1.reshape(EF // 2, 2, D, DF).transpose(0, 2, 1, 3)
          .reshape(EF // 2, D, 2 * DF).astype(bf16))
    w2 = W2.reshape(EF // 2, 2 * DF, D).astype(bf16)

    blk_tok = pl.BlockSpec((BLK, D), lambda i: (i, 0))
    blk_tab = pl.BlockSpec((BLK, DH), lambda i: (i, 0))
    blk_row = pl.BlockSpec((1, D), lambda i: (0, 0))

    out = pl.pallas_call(
        _body,
        grid=(NBLK,),
        in_specs=[blk_tok, blk_row, blk_row,
                  blk_tab, blk_tab, _full((DH, H * DH)),
                  _full((D, 4 * H * DH)),
                  _full((EA, D, H * DH)), _full((D, EA)),
                  _full((EA, H * DH, D)), _full((D, EA)),
                  _full((EF // 2, D, 2 * DF)), _full((EF // 2, 2 * DF, D)),
                  _full((D, EF))],
        out_specs=pl.BlockSpec((BLK, D), lambda i: (i, 0)),
        out_shape=jax.ShapeDtypeStruct((S, D), f32),
        scratch_shapes=[pltpu.VMEM((H, NBLK, DH, BLK), bf16),
                        pltpu.VMEM((H, NBLK, BLK, DH), bf16)],
    )(x, ga, gf, _COSC, _SINC, _TEYE, wqk, wv, sel_v, wo, sel_o,
      w1, w2, sel_f)

    return out.reshape(B, S, D)


# speedup vs baseline: 1.2349x; 1.2349x over previous
import jax
import jax.numpy as jnp
import numpy as np
from jax.experimental import pallas as pl
from jax.experimental.pallas import tpu as pltpu

B, S, D = 1, 2048, 768
H, DH = 12, 64
EA, KA = 8, 2
EF, KF, DF = 16, 8, 128

BLK = 512
NBLK = S // BLK

f32 = jnp.float32
bf16 = jnp.bfloat16


def _topk_gates_mask(logits, k):
    n, e_dim = logits.shape
    lj = logits[:, :, None]
    le = logits[:, None, :]
    jj = jax.lax.broadcasted_iota(jnp.int32, (n, e_dim, e_dim), 1)
    ee = jax.lax.broadcasted_iota(jnp.int32, (n, e_dim, e_dim), 2)
    gt = (lj > le).astype(f32)
    tie = jnp.logical_and(lj == le, jj < ee).astype(f32)
    rank = jnp.sum(gt + tie, axis=1)
    keep = (rank < k).astype(f32)
    return jax.nn.sigmoid(logits) * keep


def _rmsnorm(x, g, eps=1e-6):
    return x * jax.lax.rsqrt(jnp.mean(x * x, axis=-1, keepdims=True) + eps) * g


def _body(x_ref, ga_ref, gf_ref, cos_ref, sin_ref, teye_ref,
          wqk_ref, wv_ref, selv_ref, wo_ref, selo_ref, w1_ref, w2_ref, self_ref,
          out_ref, kt_scr, v_scr):
    i = pl.program_id(0)
    x = x_ref[...]
    h = _rmsnorm(x, ga_ref[...])
    hb = h.astype(bf16)

    hd = H * DH
    cs = jnp.dot(cos_ref[...], teye_ref[...], preferred_element_type=f32)
    sn = jnp.dot(sin_ref[...], teye_ref[...], preferred_element_type=f32)
    sq = jnp.dot(hb, wqk_ref[:, :2 * hd], preferred_element_type=f32)
    q = (sq[:, :hd] * cs + sq[:, hd:] * sn).astype(bf16)
    sk = jnp.dot(hb, wqk_ref[:, 2 * hd:], preferred_element_type=f32)
    k = (sk[:, :hd] * cs + sk[:, hd:] * sn).astype(bf16)

    ktf = k.T
    for hh in range(H):
        kt_scr[hh, i] = ktf[hh * DH:(hh + 1) * DH, :]

    logits = jnp.dot(h, selv_ref[...], preferred_element_type=f32)
    gates = _topk_gates_mask(logits, KA)
    acc = jnp.zeros((BLK, H * DH), f32)
    for e in range(EA):
        acc += jnp.dot(hb, wv_ref[e], preferred_element_type=f32) * gates[:, e:e + 1]
    v = acc.astype(bf16)
    for hh in range(H):
        v_scr[hh, i] = v[:, hh * DH:(hh + 1) * DH]

    rowg = i * BLK + jax.lax.broadcasted_iota(jnp.int32, (BLK, BLK), 0)
    coll = jax.lax.broadcasted_iota(jnp.int32, (BLK, BLK), 1)
    qs = [q[:, hh * DH:(hh + 1) * DH] for hh in range(H)]

    m0 = jnp.full((BLK, 1), -1e30, f32)
    l0 = jnp.zeros((BLK, 1), f32)
    a0 = jnp.zeros((BLK, DH), f32)
    G = 2
    ctx_parts = []
    for g0 in range(0, H, G):
        heads = list(range(g0, g0 + G))

        def chunk(c, carry, heads=heads):
            causal = (c * BLK + coll) <= rowg
            new = []
            for idx, hh in enumerate(heads):
                m, l, a = carry[idx]
                sc = jnp.dot(qs[hh], kt_scr[hh, c], preferred_element_type=f32)
                sc = jnp.where(causal, sc, -1e9)
                m_new = jnp.maximum(m, jnp.max(sc, axis=-1, keepdims=True))
                corr = jnp.exp(m - m_new)
                p = jnp.exp(sc - m_new)
                l_new = l * corr + jnp.sum(p, axis=-1, keepdims=True)
                a_new = a * corr + jnp.dot(p.astype(bf16), v_scr[hh, c],
                                           preferred_element_type=f32)
                new.append((m_new, l_new, a_new))
            return tuple(new)

        res = jax.lax.fori_loop(0, i + 1, chunk,
                                tuple((m0, l0, a0) for _ in heads))
        ctx_parts.extend((a / l).astype(bf16) for (m, l, a) in res)
    ctx = jnp.concatenate(ctx_parts, axis=1)

    logits_o = jnp.dot(h, selo_ref[...], preferred_element_type=f32)
    gates_o = _topk_gates_mask(logits_o, KA)
    acc = jnp.zeros((BLK, D), f32)
    for e in range(EA):
        acc += jnp.dot(ctx, wo_ref[e], preferred_element_type=f32) * gates_o[:, e:e + 1]
    x1 = x + acc

    h2 = _rmsnorm(x1, gf_ref[...])
    h2b = h2.astype(bf16)
    logits_f = jnp.dot(h2, self_ref[...], preferred_element_type=f32)
    gates_f = _topk_gates_mask(logits_f, KF)
    y = jnp.zeros((BLK, D), f32)
    lane = jax.lax.broadcasted_iota(jnp.int32, (BLK, 2 * DF), 1)
    for pp in range(EF // 2):
        mid = jnp.dot(h2b, w1_ref[pp], preferred_element_type=f32)
        mid = jnp.maximum(mid, 0.0)
        gw = jnp.where(lane < DF, gates_f[:, 2 * pp:2 * pp + 1],
                       gates_f[:, 2 * pp + 1:2 * pp + 2])
        mid = (mid * gw).astype(bf16)
        y += jnp.dot(mid, w2_ref[pp], preferred_element_type=f32)
    out_ref[...] = x1 + y


def _full(shape):
    return pl.BlockSpec(shape, lambda *_: (0,) * len(shape))


def _rope_tables():
    half = DH // 2
    pos = np.arange(S, dtype=np.float32)
    inv_freq = 1.0 / (10000.0 ** (np.arange(0, half, dtype=np.float32) / half))
    freqs = (pos[:, None] * inv_freq[None, :]).astype(np.float32)
    cos1 = np.cos(freqs, dtype=np.float32)
    sin1 = np.sin(freqs, dtype=np.float32)
    cos = np.tile(np.concatenate([cos1, cos1], axis=1), (1, H))
    sin = np.tile(np.concatenate([sin1, sin1], axis=1), (1, H))
    scale = np.float32(1.0) / np.sqrt(np.float32(DH))
    return (cos.astype(np.float32), sin.astype(np.float32),
            (cos * scale).astype(np.float32), (sin * scale).astype(np.float32))


_COS, _SIN, _COSQ, _SINQ = _rope_tables()
_COSC = _COS[:, :DH].copy()
_SINC = _SIN[:, :DH].copy()
_TEYE = np.tile(np.eye(DH, dtype=np.float32), (1, H))


def _prep_body(wq_ref, wk_ref, wv_ref, wo_ref, w1_ref, w2_ref,
               wqk_ref, wvb_ref, wob_ref, w1p_ref, w2p_ref):
    e = pl.program_id(0)
    wvb_ref[...] = wv_ref[...].astype(bf16)
    wob_ref[...] = wo_ref[...].astype(bf16)
    b1 = w1_ref[...]
    w1p_ref[0] = jnp.concatenate([b1[0], b1[1]], axis=1).astype(bf16)
    b2 = w2_ref[...]
    w2p_ref[0] = jnp.concatenate([b2[0], b2[1]], axis=0).astype(bf16)

    @pl.when(e == 0)
    def _():
        wq = wq_ref[...]
        wk = wk_ref[...]
        half = DH // 2

        def rot(w):
            pieces = []
            for hh in range(H):
                base = hh * DH
                pieces.append(-w[:, base + half:base + DH])
                pieces.append(w[:, base:base + half])
            return jnp.concatenate(pieces, axis=1)

        wqk_ref[...] = jnp.concatenate(
            [wq * 0.125, rot(wq) * 0.125, wk, rot(wk)], axis=1).astype(bf16)


def _prep_weights(Wq, Wk, Wv, Wo, W1, W2):
    return pl.pallas_call(
        _prep_body,
        grid=(EA,),
        in_specs=[_full((D, H * DH)), _full((D, H * DH)),
                  pl.BlockSpec((1, D, H * DH), lambda i: (i, 0, 0)),
                  pl.BlockSpec((1, H * DH, D), lambda i: (i, 0, 0)),
                  pl.BlockSpec((2, D, DF), lambda i: (i, 0, 0)),
                  pl.BlockSpec((2, DF, D), lambda i: (i, 0, 0))],
        out_specs=[_full((D, 4 * H * DH)),
                   pl.BlockSpec((1, D, H * DH), lambda i: (i, 0, 0)),
                   pl.BlockSpec((1, H * DH, D), lambda i: (i, 0, 0)),
                   pl.BlockSpec((1, D, 2 * DF), lambda i: (i, 0, 0)),
                   pl.BlockSpec((1, 2 * DF, D), lambda i: (i, 0, 0))],
        out_shape=[jax.ShapeDtypeStruct((D, 4 * H * DH), bf16),
                   jax.ShapeDtypeStruct((EA, D, H * DH), bf16),
                   jax.ShapeDtypeStruct((EA, H * DH, D), bf16),
                   jax.ShapeDtypeStruct((EF // 2, D, 2 * DF), bf16),
                   jax.ShapeDtypeStruct((EF // 2, 2 * DF, D), bf16)],
    )(Wq, Wk, Wv, Wo, W1, W2)


def kernel(token_stream, g_attn, g_ffn, Wq, Wk, Wv, Wo, sel_v, sel_o, W1, W2, sel_f):
    x = token_stream[0]
    ga = g_attn.reshape(1, D)
    gf = g_ffn.reshape(1, D)

    wqk, wv, wo, w1, w2 = _prep_weights(Wq, Wk, Wv, Wo, W1, W2)

    blk_tok = pl.BlockSpec((BLK, D), lambda i: (i, 0))
    blk_tab = pl.BlockSpec((BLK, DH), lambda i: (i, 0))
    blk_row = pl.BlockSpec((1, D), lambda i: (0, 0))

    out = pl.pallas_call(
        _body,
        grid=(NBLK,),
        in_specs=[blk_tok, blk_row, blk_row,
                  blk_tab, blk_tab, _full((DH, H * DH)),
                  _full((D, 4 * H * DH)),
                  _full((EA, D, H * DH)), _full((D, EA)),
                  _full((EA, H * DH, D)), _full((D, EA)),
                  _full((EF // 2, D, 2 * DF)), _full((EF // 2, 2 * DF, D)),
                  _full((D, EF))],
        out_specs=pl.BlockSpec((BLK, D), lambda i: (i, 0)),
        out_shape=jax.ShapeDtypeStruct((S, D), f32),
        scratch_shapes=[pltpu.VMEM((H, NBLK, DH, BLK), bf16),
                        pltpu.VMEM((H, NBLK, BLK, DH), bf16)],
    )(x, ga, gf, _COSC, _SINC, _TEYE, wqk, wv, sel_v, wo, sel_o,
      w1, w2, sel_f)

    return out.reshape(B, S, D)


# speedup vs baseline: 1.2518x; 1.0137x over previous
import jax
import jax.numpy as jnp
import numpy as np
from jax.experimental import pallas as pl
from jax.experimental.pallas import tpu as pltpu

B, S, D = 1, 2048, 768
H, DH = 12, 64
EA, KA = 8, 2
EF, KF, DF = 16, 8, 128

BLK = 512
NBLK = S // BLK

f32 = jnp.float32
bf16 = jnp.bfloat16


def _topk_gates_mask(logits, k):
    n, e_dim = logits.shape
    lj = logits[:, :, None]
    le = logits[:, None, :]
    jj = jax.lax.broadcasted_iota(jnp.int32, (n, e_dim, e_dim), 1)
    ee = jax.lax.broadcasted_iota(jnp.int32, (n, e_dim, e_dim), 2)
    gt = (lj > le).astype(f32)
    tie = jnp.logical_and(lj == le, jj < ee).astype(f32)
    rank = jnp.sum(gt + tie, axis=1)
    keep = (rank < k).astype(f32)
    return jax.nn.sigmoid(logits) * keep


def _rmsnorm(x, g, eps=1e-6):
    return x * jax.lax.rsqrt(jnp.mean(x * x, axis=-1, keepdims=True) + eps) * g


def _body(x_ref, ga_ref, gf_ref, cos_ref, sin_ref, teye_ref,
          wqk_ref, wv_ref, selv_ref, wo_ref, selo_ref, w1_ref, w2_ref, self_ref,
          out_ref, kt_scr, v_scr):
    i = pl.program_id(0)
    x = x_ref[...]
    h = _rmsnorm(x, ga_ref[...])
    hb = h.astype(bf16)

    hd = H * DH
    cs = jnp.dot(cos_ref[...], teye_ref[...], preferred_element_type=f32)
    sn = jnp.dot(sin_ref[...], teye_ref[...], preferred_element_type=f32)
    sq = jnp.dot(hb, wqk_ref[:, :2 * hd], preferred_element_type=f32)
    q = (sq[:, :hd] * cs + sq[:, hd:] * sn).astype(bf16)
    sk = jnp.dot(hb, wqk_ref[:, 2 * hd:], preferred_element_type=f32)
    k = (sk[:, :hd] * cs + sk[:, hd:] * sn).astype(bf16)

    ktf = k.T
    for hh in range(H):
        kt_scr[hh, i] = ktf[hh * DH:(hh + 1) * DH, :]

    logits = jnp.dot(h, selv_ref[...], preferred_element_type=f32)
    gates = _topk_gates_mask(logits, KA)
    acc = jnp.zeros((BLK, H * DH), f32)
    for e in range(EA):
        acc += jnp.dot(hb, wv_ref[e], preferred_element_type=f32) * gates[:, e:e + 1]
    v = acc.astype(bf16)
    for hh in range(H):
        v_scr[hh, i] = v[:, hh * DH:(hh + 1) * DH]

    rowg = i * BLK + jax.lax.broadcasted_iota(jnp.int32, (BLK, BLK), 0)
    coll = jax.lax.broadcasted_iota(jnp.int32, (BLK, BLK), 1)
    qs = [q[:, hh * DH:(hh + 1) * DH] for hh in range(H)]

    m0 = jnp.full((BLK, 1), -1e30, f32)
    l0 = jnp.zeros((BLK, 1), f32)
    a0 = jnp.zeros((BLK, DH), f32)
    G = 2
    ctx_parts = []
    for g0 in range(0, H, G):
        heads = list(range(g0, g0 + G))

        def chunk(c, carry, heads=heads):
            causal = (c * BLK + coll) <= rowg
            new = []
            for idx, hh in enumerate(heads):
                m, l, a = carry[idx]
                sc = jnp.dot(qs[hh], kt_scr[hh, c], preferred_element_type=f32)
                sc = jnp.where(causal, sc, -1e9)
                m_new = jnp.maximum(m, jnp.max(sc, axis=-1, keepdims=True))
                corr = jnp.exp(m - m_new)
                p = jnp.exp((sc - m_new).astype(bf16))
                l_new = l * corr + jnp.sum(p, axis=-1, keepdims=True,
                                           dtype=f32)
                a_new = a * corr + jnp.dot(p, v_scr[hh, c],
                                           preferred_element_type=f32)
                new.append((m_new, l_new, a_new))
            return tuple(new)

        res = jax.lax.fori_loop(0, i + 1, chunk,
                                tuple((m0, l0, a0) for _ in heads))
        ctx_parts.extend((a / l).astype(bf16) for (m, l, a) in res)
    ctx = jnp.concatenate(ctx_parts, axis=1)

    logits_o = jnp.dot(h, selo_ref[...], preferred_element_type=f32)
    gates_o = _topk_gates_mask(logits_o, KA)
    acc = jnp.zeros((BLK, D), f32)
    for e in range(EA):
        acc += jnp.dot(ctx, wo_ref[e], preferred_element_type=f32) * gates_o[:, e:e + 1]
    x1 = x + acc

    h2 = _rmsnorm(x1, gf_ref[...])
    h2b = h2.astype(bf16)
    logits_f = jnp.dot(h2, self_ref[...], preferred_element_type=f32)
    gates_f = _topk_gates_mask(logits_f, KF)
    y = jnp.zeros((BLK, D), f32)
    lane = jax.lax.broadcasted_iota(jnp.int32, (BLK, 2 * DF), 1)
    for pp in range(EF // 2):
        mid = jnp.dot(h2b, w1_ref[pp], preferred_element_type=f32)
        mid = jnp.maximum(mid, 0.0)
        gw = jnp.where(lane < DF, gates_f[:, 2 * pp:2 * pp + 1],
                       gates_f[:, 2 * pp + 1:2 * pp + 2])
        mid = (mid * gw).astype(bf16)
        y += jnp.dot(mid, w2_ref[pp], preferred_element_type=f32)
    out_ref[...] = x1 + y


def _full(shape):
    return pl.BlockSpec(shape, lambda *_: (0,) * len(shape))


def _rope_tables():
    half = DH // 2
    pos = np.arange(S, dtype=np.float32)
    inv_freq = 1.0 / (10000.0 ** (np.arange(0, half, dtype=np.float32) / half))
    freqs = (pos[:, None] * inv_freq[None, :]).astype(np.float32)
    cos1 = np.cos(freqs, dtype=np.float32)
    sin1 = np.sin(freqs, dtype=np.float32)
    cos = np.tile(np.concatenate([cos1, cos1], axis=1), (1, H))
    sin = np.tile(np.concatenate([sin1, sin1], axis=1), (1, H))
    scale = np.float32(1.0) / np.sqrt(np.float32(DH))
    return (cos.astype(np.float32), sin.astype(np.float32),
            (cos * scale).astype(np.float32), (sin * scale).astype(np.float32))


_COS, _SIN, _COSQ, _SINQ = _rope_tables()
_COSC = _COS[:, :DH].copy()
_SINC = _SIN[:, :DH].copy()
_TEYE = np.tile(np.eye(DH, dtype=np.float32), (1, H))


def _prep_body(wq_ref, wk_ref, wv_ref, wo_ref, w1_ref, w2_ref,
               wqk_ref, wvb_ref, wob_ref, w1p_ref, w2p_ref):
    e = pl.program_id(0)
    wvb_ref[...] = wv_ref[...].astype(bf16)
    wob_ref[...] = wo_ref[...].astype(bf16)
    b1 = w1_ref[...]
    w1p_ref[0] = jnp.concatenate([b1[0], b1[1]], axis=1).astype(bf16)
    b2 = w2_ref[...]
    w2p_ref[0] = jnp.concatenate([b2[0], b2[1]], axis=0).astype(bf16)

    @pl.when(e == 0)
    def _():
        wq = wq_ref[...]
        wk = wk_ref[...]
        half = DH // 2

        def rot(w):
            pieces = []
            for hh in range(H):
                base = hh * DH
                pieces.append(-w[:, base + half:base + DH])
                pieces.append(w[:, base:base + half])
            return jnp.concatenate(pieces, axis=1)

        wqk_ref[...] = jnp.concatenate(
            [wq * 0.125, rot(wq) * 0.125, wk, rot(wk)], axis=1).astype(bf16)


def _prep_weights(Wq, Wk, Wv, Wo, W1, W2):
    return pl.pallas_call(
        _prep_body,
        grid=(EA,),
        in_specs=[_full((D, H * DH)), _full((D, H * DH)),
                  pl.BlockSpec((1, D, H * DH), lambda i: (i, 0, 0)),
                  pl.BlockSpec((1, H * DH, D), lambda i: (i, 0, 0)),
                  pl.BlockSpec((2, D, DF), lambda i: (i, 0, 0)),
                  pl.BlockSpec((2, DF, D), lambda i: (i, 0, 0))],
        out_specs=[_full((D, 4 * H * DH)),
                   pl.BlockSpec((1, D, H * DH), lambda i: (i, 0, 0)),
                   pl.BlockSpec((1, H * DH, D), lambda i: (i, 0, 0)),
                   pl.BlockSpec((1, D, 2 * DF), lambda i: (i, 0, 0)),
                   pl.BlockSpec((1, 2 * DF, D), lambda i: (i, 0, 0))],
        out_shape=[jax.ShapeDtypeStruct((D, 4 * H * DH), bf16),
                   jax.ShapeDtypeStruct((EA, D, H * DH), bf16),
                   jax.ShapeDtypeStruct((EA, H * DH, D), bf16),
                   jax.ShapeDtypeStruct((EF // 2, D, 2 * DF), bf16),
                   jax.ShapeDtypeStruct((EF // 2, 2 * DF, D), bf16)],
    )(Wq, Wk, Wv, Wo, W1, W2)


def kernel(token_stream, g_attn, g_ffn, Wq, Wk, Wv, Wo, sel_v, sel_o, W1, W2, sel_f):
    x = token_stream[0]
    ga = g_attn.reshape(1, D)
    gf = g_ffn.reshape(1, D)

    wqk, wv, wo, w1, w2 = _prep_weights(Wq, Wk, Wv, Wo, W1, W2)

    blk_tok = pl.BlockSpec((BLK, D), lambda i: (i, 0))
    blk_tab = pl.BlockSpec((BLK, DH), lambda i: (i, 0))
    blk_row = pl.BlockSpec((1, D), lambda i: (0, 0))

    out = pl.pallas_call(
        _body,
        grid=(NBLK,),
        in_specs=[blk_tok, blk_row, blk_row,
                  blk_tab, blk_tab, _full((DH, H * DH)),
                  _full((D, 4 * H * DH)),
                  _full((EA, D, H * DH)), _full((D, EA)),
                  _full((EA, H * DH, D)), _full((D, EA)),
                  _full((EF // 2, D, 2 * DF)), _full((EF // 2, 2 * DF, D)),
                  _full((D, EF))],
        out_specs=pl.BlockSpec((BLK, D), lambda i: (i, 0)),
        out_shape=jax.ShapeDtypeStruct((S, D), f32),
        scratch_shapes=[pltpu.VMEM((H, NBLK, DH, BLK), bf16),
                        pltpu.VMEM((H, NBLK, BLK, DH), bf16)],
    )(x, ga, gf, _COSC, _SINC, _TEYE, wqk, wv, sel_v, wo, sel_o,
      w1, w2, sel_f)

    return out.reshape(B, S, D)
